# initial kernel scaffold (unmeasured)
import jax
import jax.numpy as jnp
from jax import lax
from jax.experimental import pallas as pl
from jax.experimental.pallas import tpu as pltpu


def kernel(
    x,
):
    def body(*refs):
        pass

    out_shape = jax.ShapeDtypeStruct(..., jnp.float32)
    return pl.pallas_call(body, out_shape=out_shape)(...)


# baseline (device time: 596150 ns/iter reference)
import jax
import jax.numpy as jnp
from jax import lax
from jax.experimental import pallas as pl
from jax.experimental.pallas import tpu as pltpu

N_DEV = 4
M, N = 8192, 1024
N_HOPS = 2 * (N_DEV - 1)

RINGS = ((1, 0, M),)
NR = len(RINGS)
CK = RINGS[0][2] // N_DEV
assert all(rows % N_DEV == 0 and rows // N_DEV == CK for _, _, rows in RINGS)


def _mod4(v):
    return lax.rem(v + 2 * N_DEV, N_DEV)


def kernel(x):
    def body(x_hbm, out_ref, comm_ref, xstage, send_sems, recv_sems,
             xdma_sems, credit_sems):
        my = lax.axis_index("i")
        left = _mod4(my - 1)
        right = _mod4(my + 1)

        barrier = pltpu.get_barrier_semaphore()
        for nbr in (left, right):
            pl.semaphore_signal(barrier, inc=1, device_id=(nbr,),
                                device_id_type=pl.DeviceIdType.MESH)
        pl.semaphore_wait(barrier, 2)

        for h in range(N_HOPS):
            send_slot, recv_slot = h % 2, (h + 1) % 2
            rdmas = []
            xcopies = []
            for r, (dirn, base, rows) in enumerate(RINGS):
                down = _mod4(my + dirn)

                if h >= 1:
                    pl.semaphore_wait(credit_sems.at[r, h], 1)

                if h == 0:
                    c_send = _mod4(my)
                    src = x_hbm.at[pl.ds(base + c_send * CK, CK), :]
                else:
                    src = comm_ref.at[r, send_slot]
                rdma = pltpu.make_async_remote_copy(
                    src_ref=src,
                    dst_ref=comm_ref.at[r, recv_slot],
                    send_sem=send_sems.at[r, h],
                    recv_sem=recv_sems.at[r, h],
                    device_id=(down,),
                    device_id_type=pl.DeviceIdType.MESH,
                )
                rdma.start()
                rdmas.append(rdma)

                if h < N_DEV - 1:
                    c_recv = _mod4(my - dirn * (h + 1))
                    xcopy = pltpu.make_async_copy(
                        x_hbm.at[pl.ds(base + c_recv * CK, CK), :],
                        xstage.at[r],
                        xdma_sems.at[r],
                    )
                    xcopy.start()
                    xcopies.append(xcopy)

            for r, (dirn, base, rows) in enumerate(RINGS):
                up = _mod4(my - dirn)
                rdmas[r].wait()
                if h < N_DEV - 1:
                    xcopies[r].wait()
                    comm_ref[r, recv_slot] = (
                        comm_ref[r, recv_slot] + xstage[r]
                    )
                    if h == N_DEV - 2:
                        c_own = _mod4(my + dirn)
                        out_ref[pl.ds(base + c_own * CK, CK), :] = (
                            comm_ref[r, recv_slot]
                        )
                else:
                    c_recv = _mod4(my - dirn * (h - (N_DEV - 1)))
                    out_ref[pl.ds(base + c_recv * CK, CK), :] = (
                        comm_ref[r, recv_slot]
                    )
                if h < N_HOPS - 1:
                    pl.semaphore_signal(credit_sems.at[r, h + 1], inc=1,
                                        device_id=(up,),
                                        device_id_type=pl.DeviceIdType.MESH)

    return pl.pallas_call(
        body,
        out_shape=jax.ShapeDtypeStruct((M, N), jnp.float32),
        in_specs=[pl.BlockSpec(memory_space=pl.ANY)],
        out_specs=pl.BlockSpec(memory_space=pltpu.VMEM),
        scratch_shapes=[
            pltpu.VMEM((NR, 2, CK, N), jnp.float32),
            pltpu.VMEM((NR, CK, N), jnp.float32),
            pltpu.SemaphoreType.DMA((NR, N_HOPS)),
            pltpu.SemaphoreType.DMA((NR, N_HOPS)),
            pltpu.SemaphoreType.DMA((NR,)),
            pltpu.SemaphoreType.REGULAR((NR, N_HOPS)),
        ],
        compiler_params=pltpu.CompilerParams(
            collective_id=0,
            vmem_limit_bytes=64 * 1024 * 1024,
        ),
    )(x)


# device time: 309288 ns/iter; 1.9275x vs baseline; 1.9275x over previous
import jax
import jax.numpy as jnp
from jax import lax
from jax.experimental import pallas as pl
from jax.experimental.pallas import tpu as pltpu

N_DEV = 4
M, N = 8192, 1024
N_HOPS = 2 * (N_DEV - 1)
RS = N_DEV - 1

RINGS = ((1, 0, M // 2), (-1, M // 2, M // 2))
NR = len(RINGS)
CK = RINGS[0][2] // N_DEV
S = 4
SB = CK // S


def _mod4(v):
    return lax.rem(v + 2 * N_DEV, N_DEV)


def kernel(x):
    def body(x_hbm, out_hbm, outv, comm_ref, send_sems, recv_sems,
             fill_sems, credit_sems, store_sems):
        my = lax.axis_index("i")
        left = _mod4(my - 1)
        right = _mod4(my + 1)

        def crow(r, c):
            return RINGS[r][1] + c * CK

        def c_recv(r, h):
            dirn = RINGS[r][0]
            if h < RS:
                return _mod4(my - dirn * (h + 1))
            return _mod4(my - dirn * (h - RS))

        def fill_copy(r, f):
            c = c_recv(r, f) if f < RS else _mod4(my)
            row = crow(r, c)
            return pltpu.make_async_copy(
                x_hbm.at[pl.ds(row, CK), :],
                outv.at[pl.ds(row, CK), :],
                fill_sems.at[r, f],
            )

        def store_copy(r, p, j):
            c = c_recv(r, p + 2)
            row = crow(r, c) + j * SB
            return pltpu.make_async_copy(
                outv.at[pl.ds(row, SB), :],
                out_hbm.at[pl.ds(row, SB), :],
                store_sems.at[r, p, j],
            )

        def rdma_desc(r, h, j):
            dirn = RINGS[r][0]
            send_slot, recv_slot = h % 2, (h + 1) % 2
            if h == 0:
                row = crow(r, _mod4(my)) + j * SB
                src = outv.at[pl.ds(row, SB), :]
                dst = comm_ref.at[r, recv_slot, pl.ds(j * SB, SB), :]
            elif h < RS:
                src = comm_ref.at[r, send_slot, pl.ds(j * SB, SB), :]
                dst = comm_ref.at[r, recv_slot, pl.ds(j * SB, SB), :]
            else:
                row = crow(r, c_recv(r, h - 1)) + j * SB
                src = outv.at[pl.ds(row, SB), :]
                dst = outv.at[pl.ds(row, SB), :]
            return pltpu.make_async_remote_copy(
                src_ref=src,
                dst_ref=dst,
                send_sem=send_sems.at[r, h, j],
                recv_sem=recv_sems.at[r, h, j],
                device_id=(_mod4(my + dirn),),
                device_id_type=pl.DeviceIdType.MESH,
            )

        for r in range(NR):
            for f in range(RS + 1):
                fill_copy(r, f).start()

        barrier = pltpu.get_barrier_semaphore()
        for nbr in (left, right):
            pl.semaphore_signal(barrier, inc=1, device_id=(nbr,),
                                device_id_type=pl.DeviceIdType.MESH)
        pl.semaphore_wait(barrier, 2)

        for h in range(N_HOPS + 1):
            for j in range(S):
                for r, (dirn, base, rows) in enumerate(RINGS):
                    up = _mod4(my - dirn)
                    if h >= 1:
                        prev = rdma_desc(r, h - 1, j)
                        prev.wait_send()
                        if h == 2:
                            pl.semaphore_signal(
                                credit_sems.at[r, j], inc=1,
                                device_id=(up,),
                                device_id_type=pl.DeviceIdType.MESH)
                        prev.wait_recv()
                        if h - 1 < RS:
                            slot = h % 2
                            c = c_recv(r, h - 1)
                            row = crow(r, c) + j * SB
                            if j == 0:
                                fill_copy(r, h - 1).wait()
                            acc = (comm_ref[r, slot, pl.ds(j * SB, SB), :]
                                   + outv[pl.ds(row, SB), :])
                            if h - 1 == RS - 1:
                                outv[pl.ds(row, SB), :] = acc
                            else:
                                comm_ref[r, slot, pl.ds(j * SB, SB), :] = acc
                        if h >= RS:
                            store_copy(r, h - RS, j).start()
                    if h <= N_HOPS - 1:
                        if h == 0 and j == 0:
                            fill_copy(r, RS).wait()
                        if h == 2:
                            pl.semaphore_wait(credit_sems.at[r, j], 1)
                        rdma_desc(r, h, j).start()

        for r in range(NR):
            for p in range(RS + 1):
                for j in range(S):
                    store_copy(r, p, j).wait()

    return pl.pallas_call(
        body,
        out_shape=jax.ShapeDtypeStruct((M, N), jnp.float32),
        in_specs=[pl.BlockSpec(memory_space=pl.ANY)],
        out_specs=pl.BlockSpec(memory_space=pl.ANY),
        scratch_shapes=[
            pltpu.VMEM((M, N), jnp.float32),
            pltpu.VMEM((NR, 2, CK, N), jnp.float32),
            pltpu.SemaphoreType.DMA((NR, N_HOPS, S)),
            pltpu.SemaphoreType.DMA((NR, N_HOPS, S)),
            pltpu.SemaphoreType.DMA((NR, RS + 1)),
            pltpu.SemaphoreType.REGULAR((NR, S)),
            pltpu.SemaphoreType.DMA((NR, RS + 1, S)),
        ],
        compiler_params=pltpu.CompilerParams(
            collective_id=0,
            vmem_limit_bytes=64 * 1024 * 1024,
        ),
    )(x)
